# baseline (speedup 1.0000x reference)
import jax
import jax.numpy as jnp
from jax.experimental import pallas as pl
from jax.experimental.pallas import tpu as pltpu

_D = 1024
_HID = 256
_E = 8
_K = 3
_FF = 1024


def _router_body(x_ref, w1_ref, b1_ref, w2_ref, b2_ref,
                 probs_ref, tkw_ref, tki_ref):
    pooled = jnp.mean(x_ref[...], axis=1)
    h = jnp.tanh(
        jnp.dot(pooled, w1_ref[...], preferred_element_type=jnp.float32)
        + b1_ref[...])
    logits = (jnp.dot(h, w2_ref[...], preferred_element_type=jnp.float32)
              + b2_ref[...])
    m = jnp.max(logits, axis=-1, keepdims=True)
    ex = jnp.exp(logits - m)
    probs = ex / jnp.sum(ex, axis=-1, keepdims=True)
    probs_ref[...] = probs

    ids = jax.lax.broadcasted_iota(jnp.int32, probs.shape, 1)
    p = probs
    ws, idxs = [], []
    for _ in range(_K):
        mk = jnp.max(p, axis=-1)
        sel = jnp.min(jnp.where(p == mk[:, None], ids, _E), axis=-1)
        ws.append(mk)
        idxs.append(sel)
        p = jnp.where(ids == sel[:, None], -1.0, p)
    tkw_ref[...] = jnp.stack(ws, axis=1)
    tki_ref[...] = jnp.stack(idxs, axis=1)


_TS = 512


def _moe_body(idx_ref, w_ref, x_ref, w1_ref, b1_ref, w2_ref, b2_ref,
              out_ref, xsave_ref, acc_ref, w1b_ref, w2b_ref):
    b = pl.program_id(0)
    k = pl.program_id(1)
    s = pl.program_id(2)
    w = w_ref[b, k]
    rows = pl.ds(s * _TS, _TS)

    @pl.when(k == 0)
    def _():
        xsave_ref[rows, :] = x_ref[0].astype(jnp.bfloat16)

    @pl.when(s == 0)
    def _():
        w1b_ref[...] = w1_ref[0].astype(jnp.bfloat16)
        w2b_ref[...] = w2_ref[0].astype(jnp.bfloat16)

    xv = xsave_ref[rows, :]
    h = jnp.tanh(
        jnp.dot(xv, w1b_ref[...], preferred_element_type=jnp.float32)
        + b1_ref[0])
    y = (jnp.dot(h.astype(jnp.bfloat16), w2b_ref[...],
                 preferred_element_type=jnp.float32)
         + b2_ref[0])

    @pl.when(k == 0)
    def _():
        acc_ref[rows, :] = x_ref[0] + w * y

    @pl.when((k > 0) & (k < _K - 1))
    def _():
        acc_ref[rows, :] = acc_ref[rows, :] + w * y

    @pl.when(k == _K - 1)
    def _():
        out_ref[0] = acc_ref[rows, :] + w * y


def kernel(x, router_w1, router_b1, router_w2, router_b2,
           zone_w1, zone_b1, zone_w2, zone_b2):
    B, S, D = x.shape

    probs, tkw, tki = pl.pallas_call(
        _router_body,
        out_shape=[
            jax.ShapeDtypeStruct((B, _E), jnp.float32),
            jax.ShapeDtypeStruct((B, _K), jnp.float32),
            jax.ShapeDtypeStruct((B, _K), jnp.int32),
        ],
    )(x, router_w1, router_b1, router_w2, router_b2)

    grid_spec = pltpu.PrefetchScalarGridSpec(
        num_scalar_prefetch=2,
        grid=(B, _K, S // _TS),
        in_specs=[
            pl.BlockSpec((1, _TS, D),
                         lambda b, k, s, idx, w:
                         (b, jnp.where(k == 0, s, 0), 0)),
            pl.BlockSpec((1, D, _FF),
                         lambda b, k, s, idx, w: (idx[b, k], 0, 0)),
            pl.BlockSpec((1, 1, _FF),
                         lambda b, k, s, idx, w: (idx[b, k], 0, 0)),
            pl.BlockSpec((1, _FF, D),
                         lambda b, k, s, idx, w: (idx[b, k], 0, 0)),
            pl.BlockSpec((1, 1, D),
                         lambda b, k, s, idx, w: (idx[b, k], 0, 0)),
        ],
        out_specs=pl.BlockSpec(
            (1, _TS, D),
            lambda b, k, s, idx, w: (b, jnp.where(k == _K - 1, s, 0), 0)),
        scratch_shapes=[
            pltpu.VMEM((S, D), jnp.bfloat16),
            pltpu.VMEM((S, D), jnp.float32),
            pltpu.VMEM((D, _FF), jnp.bfloat16),
            pltpu.VMEM((_FF, D), jnp.bfloat16),
        ],
    )
    out = pl.pallas_call(
        _moe_body,
        grid_spec=grid_spec,
        out_shape=jax.ShapeDtypeStruct((B, S, D), jnp.float32),
    )(tki, tkw, x, zone_w1, zone_b1.reshape(_E, 1, _FF),
      zone_w2, zone_b2.reshape(_E, 1, D))

    return (out, probs)


# speedup vs baseline: 1.1085x; 1.1085x over previous
import jax
import jax.numpy as jnp
from jax.experimental import pallas as pl
from jax.experimental.pallas import tpu as pltpu

_D = 1024
_HID = 256
_E = 8
_K = 3
_FF = 1024


def _router_body(x_ref, w1_ref, b1_ref, w2_ref, b2_ref,
                 probs_ref, tkw_ref, tki_ref):
    pooled = jnp.mean(x_ref[...], axis=1)
    h = jnp.tanh(
        jnp.dot(pooled, w1_ref[...], preferred_element_type=jnp.float32)
        + b1_ref[...])
    logits = (jnp.dot(h, w2_ref[...], preferred_element_type=jnp.float32)
              + b2_ref[...])
    m = jnp.max(logits, axis=-1, keepdims=True)
    ex = jnp.exp(logits - m)
    probs = ex / jnp.sum(ex, axis=-1, keepdims=True)
    probs_ref[...] = probs

    ids = jax.lax.broadcasted_iota(jnp.int32, probs.shape, 1)
    p = probs
    ws, idxs = [], []
    for _ in range(_K):
        mk = jnp.max(p, axis=-1)
        sel = jnp.min(jnp.where(p == mk[:, None], ids, _E), axis=-1)
        ws.append(mk)
        idxs.append(sel)
        p = jnp.where(ids == sel[:, None], -1.0, p)
    tkw_ref[...] = jnp.stack(ws, axis=1)
    tki_ref[...] = jnp.stack(idxs, axis=1)


_TS = 1024


def _moe_body(idx_ref, w_ref, x_ref, w1_ref, b1_ref, w2_ref, b2_ref,
              out_ref, acc_ref, w1b_ref, w2b_ref):
    b = pl.program_id(0)
    k = pl.program_id(1)
    s = pl.program_id(2)
    w = w_ref[b, k]
    rows = pl.ds(s * _TS, _TS)

    @pl.when(s == 0)
    def _():
        w1b_ref[...] = w1_ref[0].astype(jnp.bfloat16)
        w2b_ref[...] = w2_ref[0].astype(jnp.bfloat16)

    xv = x_ref[0]
    h = jnp.tanh(
        jnp.dot(xv.astype(jnp.bfloat16), w1b_ref[...],
                preferred_element_type=jnp.float32)
        + b1_ref[0])
    y = (jnp.dot(h.astype(jnp.bfloat16), w2b_ref[...],
                 preferred_element_type=jnp.float32)
         + b2_ref[0])

    @pl.when(k == 0)
    def _():
        acc_ref[rows, :] = w * y

    @pl.when((k > 0) & (k < _K - 1))
    def _():
        acc_ref[rows, :] = acc_ref[rows, :] + w * y

    @pl.when(k == _K - 1)
    def _():
        out_ref[0] = xv + acc_ref[rows, :] + w * y


def kernel(x, router_w1, router_b1, router_w2, router_b2,
           zone_w1, zone_b1, zone_w2, zone_b2):
    B, S, D = x.shape

    probs, tkw, tki = pl.pallas_call(
        _router_body,
        out_shape=[
            jax.ShapeDtypeStruct((B, _E), jnp.float32),
            jax.ShapeDtypeStruct((B, _K), jnp.float32),
            jax.ShapeDtypeStruct((B, _K), jnp.int32),
        ],
    )(x, router_w1, router_b1, router_w2, router_b2)

    grid_spec = pltpu.PrefetchScalarGridSpec(
        num_scalar_prefetch=2,
        grid=(B, _K, S // _TS),
        in_specs=[
            pl.BlockSpec((1, _TS, D),
                         lambda b, k, s, idx, w: (b, s, 0)),
            pl.BlockSpec((1, D, _FF),
                         lambda b, k, s, idx, w: (idx[b, k], 0, 0)),
            pl.BlockSpec((1, 1, _FF),
                         lambda b, k, s, idx, w: (idx[b, k], 0, 0)),
            pl.BlockSpec((1, _FF, D),
                         lambda b, k, s, idx, w: (idx[b, k], 0, 0)),
            pl.BlockSpec((1, 1, D),
                         lambda b, k, s, idx, w: (idx[b, k], 0, 0)),
        ],
        out_specs=pl.BlockSpec(
            (1, _TS, D),
            lambda b, k, s, idx, w: (b, jnp.where(k == _K - 1, s, 0), 0)),
        scratch_shapes=[
            pltpu.VMEM((S, D), jnp.float32),
            pltpu.VMEM((D, _FF), jnp.bfloat16),
            pltpu.VMEM((_FF, D), jnp.bfloat16),
        ],
    )
    out = pl.pallas_call(
        _moe_body,
        grid_spec=grid_spec,
        out_shape=jax.ShapeDtypeStruct((B, S, D), jnp.float32),
    )(tki, tkw, x, zone_w1, zone_b1.reshape(_E, 1, _FF),
      zone_w2, zone_b2.reshape(_E, 1, D))

    return (out, probs)


# speedup vs baseline: 1.1522x; 1.0395x over previous
import jax
import jax.numpy as jnp
from jax.experimental import pallas as pl
from jax.experimental.pallas import tpu as pltpu

_D = 1024
_HID = 256
_E = 8
_K = 3
_FF = 1024


def _router_body(x_ref, w1_ref, b1_ref, w2_ref, b2_ref,
                 probs_ref, tkw_ref, tki_ref):
    pooled = jnp.mean(x_ref[...], axis=1)
    h = jnp.tanh(
        jnp.dot(pooled, w1_ref[...], preferred_element_type=jnp.float32)
        + b1_ref[...])
    logits = (jnp.dot(h, w2_ref[...], preferred_element_type=jnp.float32)
              + b2_ref[...])
    m = jnp.max(logits, axis=-1, keepdims=True)
    ex = jnp.exp(logits - m)
    probs = ex / jnp.sum(ex, axis=-1, keepdims=True)
    probs_ref[...] = probs

    ids = jax.lax.broadcasted_iota(jnp.int32, probs.shape, 1)
    p = probs
    ws, idxs = [], []
    for _ in range(_K):
        mk = jnp.max(p, axis=-1)
        sel = jnp.min(jnp.where(p == mk[:, None], ids, _E), axis=-1)
        ws.append(mk)
        idxs.append(sel)
        p = jnp.where(ids == sel[:, None], -1.0, p)
    tkw_ref[...] = jnp.stack(ws, axis=1)
    tki_ref[...] = jnp.stack(idxs, axis=1)


_TS = 1024


def _moe_body(idx_ref, w_ref, x_ref, w1_ref, b1_ref, w2_ref, b2_ref,
              out_ref, acc_ref, xb_ref, w1b_ref, w2b_ref):
    b = pl.program_id(0)
    k = pl.program_id(1)
    s = pl.program_id(2)
    w = w_ref[b, k]
    rows = pl.ds(s * _TS, _TS)

    @pl.when(s == 0)
    def _():
        w1b_ref[...] = w1_ref[0].astype(jnp.bfloat16)
        w2b_ref[...] = w2_ref[0].astype(jnp.bfloat16)

    def ffn(xb16):
        h = jnp.tanh(
            jnp.dot(xb16, w1b_ref[...], preferred_element_type=jnp.float32)
            + b1_ref[0])
        return (jnp.dot(h.astype(jnp.bfloat16), w2b_ref[...],
                        preferred_element_type=jnp.float32)
                + b2_ref[0])

    @pl.when(k == 0)
    def _():
        xv = x_ref[0]
        xc = xv.astype(jnp.bfloat16)
        xb_ref[rows, :] = xc
        acc_ref[rows, :] = xv + w * ffn(xc)

    @pl.when((k > 0) & (k < _K - 1))
    def _():
        acc_ref[rows, :] = acc_ref[rows, :] + w * ffn(xb_ref[rows, :])

    @pl.when(k == _K - 1)
    def _():
        out_ref[0] = acc_ref[rows, :] + w * ffn(xb_ref[rows, :])


def kernel(x, router_w1, router_b1, router_w2, router_b2,
           zone_w1, zone_b1, zone_w2, zone_b2):
    B, S, D = x.shape

    probs, tkw, tki = pl.pallas_call(
        _router_body,
        out_shape=[
            jax.ShapeDtypeStruct((B, _E), jnp.float32),
            jax.ShapeDtypeStruct((B, _K), jnp.float32),
            jax.ShapeDtypeStruct((B, _K), jnp.int32),
        ],
    )(x, router_w1, router_b1, router_w2, router_b2)

    grid_spec = pltpu.PrefetchScalarGridSpec(
        num_scalar_prefetch=2,
        grid=(B, _K, S // _TS),
        in_specs=[
            pl.BlockSpec((1, _TS, D),
                         lambda b, k, s, idx, w:
                         (b, jnp.where(k == 0, s, 0), 0)),
            pl.BlockSpec((1, D, _FF),
                         lambda b, k, s, idx, w: (idx[b, k], 0, 0)),
            pl.BlockSpec((1, 1, _FF),
                         lambda b, k, s, idx, w: (idx[b, k], 0, 0)),
            pl.BlockSpec((1, _FF, D),
                         lambda b, k, s, idx, w: (idx[b, k], 0, 0)),
            pl.BlockSpec((1, 1, D),
                         lambda b, k, s, idx, w: (idx[b, k], 0, 0)),
        ],
        out_specs=pl.BlockSpec(
            (1, _TS, D),
            lambda b, k, s, idx, w: (b, jnp.where(k == _K - 1, s, 0), 0)),
        scratch_shapes=[
            pltpu.VMEM((S, D), jnp.float32),
            pltpu.VMEM((S, D), jnp.bfloat16),
            pltpu.VMEM((D, _FF), jnp.bfloat16),
            pltpu.VMEM((_FF, D), jnp.bfloat16),
        ],
    )
    out = pl.pallas_call(
        _moe_body,
        grid_spec=grid_spec,
        out_shape=jax.ShapeDtypeStruct((B, S, D), jnp.float32),
    )(tki, tkw, x, zone_w1, zone_b1.reshape(_E, 1, _FF),
      zone_w2, zone_b2.reshape(_E, 1, D))

    return (out, probs)


# speedup vs baseline: 1.1551x; 1.0025x over previous
import jax
import jax.numpy as jnp
from jax.experimental import pallas as pl
from jax.experimental.pallas import tpu as pltpu

_D = 1024
_HID = 256
_E = 8
_K = 3
_FF = 1024


def _router_body(x_ref, w1_ref, b1_ref, w2_ref, b2_ref,
                 probs_ref, tkw_ref, tki_ref):
    pooled = jnp.mean(x_ref[...], axis=1)
    h = jnp.tanh(
        jnp.dot(pooled, w1_ref[...], preferred_element_type=jnp.float32)
        + b1_ref[...])
    logits = (jnp.dot(h, w2_ref[...], preferred_element_type=jnp.float32)
              + b2_ref[...])
    m = jnp.max(logits, axis=-1, keepdims=True)
    ex = jnp.exp(logits - m)
    probs = ex / jnp.sum(ex, axis=-1, keepdims=True)
    probs_ref[...] = probs

    ids = jax.lax.broadcasted_iota(jnp.int32, probs.shape, 1)
    p = probs
    ws, idxs = [], []
    for _ in range(_K):
        mk = jnp.max(p, axis=-1)
        sel = jnp.min(jnp.where(p == mk[:, None], ids, _E), axis=-1)
        ws.append(mk)
        idxs.append(sel)
        p = jnp.where(ids == sel[:, None], -1.0, p)
    tkw_ref[...] = jnp.stack(ws, axis=1)
    tki_ref[...] = jnp.stack(idxs, axis=1)


_TS = 1024


def _moe_body(idx_ref, w_ref, x_ref, w1_ref, b1_ref, w2_ref, b2_ref,
              out_ref, acc_ref, xb_ref, w1b_ref, w2b_ref):
    b = pl.program_id(0)
    k = pl.program_id(1)
    s = pl.program_id(2)
    w = w_ref[b, k]
    rows = pl.ds(s * _TS, _TS)

    @pl.when(s == 0)
    def _():
        w1b_ref[...] = w1_ref[0].astype(jnp.bfloat16)
        w2b_ref[...] = (w * w2_ref[0]).astype(jnp.bfloat16)

    def ffn(xb16):
        h = jnp.tanh(
            jnp.dot(xb16, w1b_ref[...],
                    preferred_element_type=jnp.float32)
            + b1_ref[0]).astype(jnp.bfloat16)
        y = (jnp.dot(h, w2b_ref[...],
                     preferred_element_type=jnp.float32)
             + w * b2_ref[0])
        return y.astype(jnp.bfloat16)

    @pl.when(k == 0)
    def _():
        xc = x_ref[0].astype(jnp.bfloat16)
        xb_ref[rows, :] = xc
        acc_ref[rows, :] = xc + ffn(xc)

    @pl.when((k > 0) & (k < _K - 1))
    def _():
        acc_ref[rows, :] = acc_ref[rows, :] + ffn(xb_ref[rows, :])

    @pl.when(k == _K - 1)
    def _():
        out_ref[0] = (acc_ref[rows, :]
                      + ffn(xb_ref[rows, :])).astype(jnp.float32)


def kernel(x, router_w1, router_b1, router_w2, router_b2,
           zone_w1, zone_b1, zone_w2, zone_b2):
    B, S, D = x.shape

    probs, tkw, tki = pl.pallas_call(
        _router_body,
        out_shape=[
            jax.ShapeDtypeStruct((B, _E), jnp.float32),
            jax.ShapeDtypeStruct((B, _K), jnp.float32),
            jax.ShapeDtypeStruct((B, _K), jnp.int32),
        ],
    )(x, router_w1, router_b1, router_w2, router_b2)

    grid_spec = pltpu.PrefetchScalarGridSpec(
        num_scalar_prefetch=2,
        grid=(B, _K, S // _TS),
        in_specs=[
            pl.BlockSpec((1, _TS, D),
                         lambda b, k, s, idx, w:
                         (b, jnp.where(k == 0, s, 0), 0)),
            pl.BlockSpec((1, D, _FF),
                         lambda b, k, s, idx, w: (idx[b, k], 0, 0)),
            pl.BlockSpec((1, 1, _FF),
                         lambda b, k, s, idx, w: (idx[b, k], 0, 0)),
            pl.BlockSpec((1, _FF, D),
                         lambda b, k, s, idx, w: (idx[b, k], 0, 0)),
            pl.BlockSpec((1, 1, D),
                         lambda b, k, s, idx, w: (idx[b, k], 0, 0)),
        ],
        out_specs=pl.BlockSpec(
            (1, _TS, D),
            lambda b, k, s, idx, w: (b, jnp.where(k == _K - 1, s, 0), 0)),
        scratch_shapes=[
            pltpu.VMEM((S, D), jnp.bfloat16),
            pltpu.VMEM((S, D), jnp.bfloat16),
            pltpu.VMEM((D, _FF), jnp.bfloat16),
            pltpu.VMEM((_FF, D), jnp.bfloat16),
        ],
    )
    out = pl.pallas_call(
        _moe_body,
        grid_spec=grid_spec,
        out_shape=jax.ShapeDtypeStruct((B, S, D), jnp.float32),
    )(tki, tkw, x, zone_w1, zone_b1.reshape(_E, 1, _FF),
      zone_w2, zone_b2.reshape(_E, 1, D))

    return (out, probs)
